# baseline (speedup 1.0000x reference)
import jax
import jax.numpy as jnp
from jax.experimental import pallas as pl
from jax.experimental.pallas import tpu as pltpu

_NEG_INF = float("-inf")
_H = 224
_W = 224
_WPAD = 256


def _conv_stack(xp, c_in):
    pieces = []
    for dy in range(3):
        rows = xp[:, dy:dy + _H, :]
        for dx in range(3):
            if dx == 0:
                pieces.append(rows)
            else:
                pieces.append(jnp.roll(rows, -dx, axis=2))
    return jnp.concatenate(pieces, axis=0)


def _layer(xp, wm_ref, b_ref, c_in):
    stack = _conv_stack(xp, c_in)
    y = jax.lax.dot_general(
        wm_ref[...], stack,
        dimension_numbers=(((1,), (0,)), ((), ())),
        preferred_element_type=jnp.float32)
    return jax.nn.relu(y + b_ref[...])


def _fused_kernel(x_ref, mask_ref, wm1_ref, b1_ref, wm2_ref, b2_ref,
                  wm3_ref, b3_ref, wl_ref, bl_ref,
                  sparse_ref, probs_ref, xp1, xp2, xp3, pool_acc):
    b = pl.program_id(0)
    nb = pl.num_programs(0)

    @pl.when(b == 0)
    def _init():
        xp1[...] = jnp.zeros_like(xp1)
        xp2[...] = jnp.zeros_like(xp2)
        xp3[...] = jnp.zeros_like(xp3)
        pool_acc[...] = jnp.zeros_like(pool_acc)

    xp1[:, 1:1 + _H, 1:1 + _W] = x_ref[0]
    y1 = _layer(xp1[...], wm1_ref, b1_ref, 3)
    xp2[:, 1:1 + _H, 1:1 + _W] = y1[:, :, :_W]
    y2 = _layer(xp2[...], wm2_ref, b2_ref, 6)
    xp3[:, 1:1 + _H, 1:1 + _W] = y2[:, :, :_W]
    y3 = _layer(xp3[...], wm3_ref, b3_ref, 12)

    pooled = jnp.sum(y3[:, :, :_W], axis=(1, 2)) * (1.0 / (_H * _W))
    lane = jax.lax.broadcasted_iota(jnp.int32, pool_acc.shape, 1)
    pool_acc[...] += jnp.where(lane == b, pooled[:, None], 0.0)

    @pl.when(b == nb - 1)
    def _route():
        logits = jax.lax.dot(wl_ref[...], pool_acc[...],
                             preferred_element_type=jnp.float32)
        masked = jnp.where(mask_ref[...] == 0, _NEG_INF,
                           logits + bl_ref[...])

        m = jnp.max(masked, axis=0, keepdims=True)
        ex = jnp.exp(masked - m)
        probs_ref[...] = ex / jnp.sum(ex, axis=0, keepdims=True)

        iota = jax.lax.broadcasted_iota(jnp.int32, masked.shape, 0)
        e1 = jnp.min(jnp.where(masked == m, iota, 99), axis=0, keepdims=True)
        sel1 = iota == e1
        l2 = jnp.where(sel1, _NEG_INF, masked)
        m2 = jnp.max(l2, axis=0, keepdims=True)
        e2 = jnp.min(jnp.where(l2 == m2, iota, 99), axis=0, keepdims=True)
        sel2 = iota == e2
        d2 = jnp.exp(m2 - m)
        denom = 1.0 + d2
        zeros = jnp.zeros_like(masked)
        sparse_ref[...] = (jnp.where(sel1, 1.0 / denom, zeros)
                           + jnp.where(sel2, d2 / denom, zeros))


def _fold(W, b, g, be, eps=1e-5):
    scale = g / jnp.sqrt(1.0 + eps)
    wm = jnp.transpose(W * scale[:, None, None, None], (0, 2, 3, 1))
    wm = wm.reshape(W.shape[0], -1)
    bias = (b * scale + be)[:, None, None]
    return wm, bias


def kernel(x, mask, W1, b1, g1, be1, W2, b2, g2, be2, W3, b3, g3, be3, Wl, bl):
    B = x.shape[0]
    wm1, bb1 = _fold(W1, b1, g1, be1)
    wm2, bb2 = _fold(W2, b2, g2, be2)
    wm3, bb3 = _fold(W3, b3, g3, be3)
    bl2 = bl[:, None]
    mask_t = mask.T

    def full(a):
        return pl.BlockSpec(a.shape, lambda b: (0,) * a.ndim)

    sparse_t, probs_t = pl.pallas_call(
        _fused_kernel,
        grid=(B,),
        in_specs=[
            pl.BlockSpec((1, 3, _H, _W), lambda b: (b, 0, 0, 0)),
            full(mask_t),
            full(wm1), full(bb1), full(wm2), full(bb2),
            full(wm3), full(bb3), full(Wl), full(bl2),
        ],
        out_specs=[
            pl.BlockSpec((16, B), lambda b: (0, 0)),
            pl.BlockSpec((16, B), lambda b: (0, 0)),
        ],
        out_shape=[
            jax.ShapeDtypeStruct((16, B), jnp.float32),
            jax.ShapeDtypeStruct((16, B), jnp.float32),
        ],
        scratch_shapes=[
            pltpu.VMEM((3, _H + 2, _WPAD), jnp.float32),
            pltpu.VMEM((6, _H + 2, _WPAD), jnp.float32),
            pltpu.VMEM((12, _H + 2, _WPAD), jnp.float32),
            pltpu.VMEM((12, 128), jnp.float32),
        ],
    )(x, mask_t, wm1, bb1, wm2, bb2, wm3, bb3, Wl, bl2)
    return (sparse_t.T, probs_t.T)
